# baseline (speedup 1.0000x reference)
import jax
import jax.numpy as jnp
from jax import lax
from jax.experimental import pallas as pl
from jax.experimental.pallas import tpu as pltpu
from jax.experimental.pallas import tpu_sc as plsc

D = 32
B = 16384
NE = 1000000
NC, NS, L = 2, 16, 16
NW = NC * NS
BPW = B // NW
CHUNKS = BPW // L

FULL_TILES = NE // 128
TAIL_COLS = NE - FULL_TILES * 128
CW = 96


def _rsqrt_newton(s):
    i = plsc.bitcast(s, jnp.int32)
    i = jnp.int32(0x5F3759DF) - lax.shift_right_logical(i, 1)
    y = plsc.bitcast(i, jnp.float32)
    half_s = 0.5 * s
    for _ in range(3):
        y = y * (1.5 - half_s * y * y)
    return y


def _detile_body(ent_hbm, tail_hbm, flat_out, buf_v, line_v, tail_v):
    wid = lax.axis_index("s") * NC + lax.axis_index("c")
    tr = lax.shift_right_logical(wid, 3)
    slot = lax.bitwise_and(wid, jnp.int32(7))
    tc0 = slot * 976 + jnp.minimum(slot, 4)
    row0 = tr * 8

    def copy_span(tc, n_tiles):
        w = n_tiles * 128
        pltpu.sync_copy(ent_hbm.at[pl.ds(row0 * 1, 8),
                                   pl.ds(tc * 128, w)],
                        buf_v.at[:, pl.ds(0, w)])
        for s in range(8):
            def destride(g, _):
                for u in range(8):
                    off = g * 128 + u * L
                    line_v[pl.ds(off, L)] = buf_v[s, pl.ds(off, L)]
                return _

            lax.fori_loop(0, n_tiles, destride, None)
            pltpu.sync_copy(
                line_v.at[pl.ds(0, w)],
                flat_out.at[pl.ds((row0 + s) * NE + tc * 128, w)])

    def chunk_body(k, _):
        copy_span(tc0 + k * CW, CW)
        return _

    lax.fori_loop(0, 10, chunk_body, None)
    copy_span(tc0 + 960, 16)
    @pl.when(slot < 4)
    def _extra():
        copy_span(tc0 + 976, 1)

    @pl.when(wid == 31)
    def _tail():
        pltpu.sync_copy(tail_hbm, tail_v)
        lanes = lax.iota(jnp.int32, L)
        for j in range(D):
            jc = jnp.full((L,), j, dtype=jnp.int32)
            for q in range(TAIL_COLS // L):
                rows = q * L + lanes
                line_v[pl.ds(q * L, L)] = plsc.load_gather(
                    tail_v, [rows, jc])
            pltpu.sync_copy(
                line_v.at[pl.ds(0, TAIL_COLS)],
                flat_out.at[pl.ds(j * NE + FULL_TILES * 128, TAIL_COLS)])


def _gather_body(flat_ent, rel_hbm, ph, pr, pt, nh, nr, nt,
                 o0, o1, o2, o3, o4, o5,
                 idx_v, eidx_v, land_v, landr_v, out_v, sem):
    wid = lax.axis_index("s") * NC + lax.axis_index("c")
    base = wid * BPW
    lanes = lax.iota(jnp.int32, L)
    diag = [lax.bitwise_and(lanes + d, jnp.int32(D - 1)) for d in range(D)]

    ent_jobs = ((ph, o0), (pt, o2), (nh, o3), (nt, o5))
    rel_jobs = ((pr, o1), (nr, o4))

    for idx_hbm, out_hbm in ent_jobs:
        pltpu.sync_copy(idx_hbm.at[pl.ds(base, BPW)], idx_v)

        def expand_body(c, _):
            chunk = idx_v[pl.ds(c * L, L)]
            for j in range(D):
                eidx_v[pl.ds(j * BPW + c * L, L)] = chunk + jnp.int32(j * NE)
            return _

        lax.fori_loop(0, CHUNKS, expand_body, None)
        pltpu.async_copy(flat_ent.at[eidx_v], land_v, sem).wait()

        def norm_body(c, _):
            cols = [land_v[pl.ds(j * BPW + c * L, L)] for j in range(D)]
            s = cols[0] * cols[0]
            for j in range(1, D):
                s = s + cols[j] * cols[j]
            y = _rsqrt_newton(jnp.maximum(s, 1e-24))
            for j in range(D):
                out_v[j, pl.ds(c * L, L)] = cols[j] * y
            return _

        lax.fori_loop(0, CHUNKS, norm_body, None)
        pltpu.sync_copy(out_v, out_hbm.at[:, pl.ds(base, BPW)])

    for idx_hbm, out_hbm in rel_jobs:
        pltpu.sync_copy(idx_hbm.at[pl.ds(base, BPW)], idx_v)

        def row_body(c, _):
            chunk = idx_v[pl.ds(c * L, L)]
            for jj in range(L):
                pltpu.async_copy(rel_hbm.at[pl.ds(chunk[jj], 1)],
                                 landr_v.at[pl.ds(c * L + jj, 1)], sem)
            return _

        lax.fori_loop(0, CHUNKS, row_body, None)
        pltpu.make_async_copy(rel_hbm.at[pl.ds(0, BPW)], landr_v, sem).wait()

        def normr_body(c, _):
            row_ids = c * L + lanes
            vals = [plsc.load_gather(landr_v, [row_ids, diag[d]])
                    for d in range(D)]
            s = vals[0] * vals[0]
            for d in range(1, D):
                s = s + vals[d] * vals[d]
            y = _rsqrt_newton(jnp.maximum(s, 1e-24))
            for d in range(D):
                plsc.store_scatter(out_v, [diag[d], row_ids], vals[d] * y)
            return _

        lax.fori_loop(0, CHUNKS, normr_body, None)
        pltpu.sync_copy(out_v, out_hbm.at[:, pl.ds(base, BPW)])


@jax.jit
def kernel(rv_ent_embeds, rel_embeds, rel_pos_hs, rel_pos_rs, rel_pos_ts,
           rel_neg_hs, rel_neg_rs, rel_neg_ts):
    mesh = plsc.VectorSubcoreMesh(core_axis_name="c", subcore_axis_name="s",
                                  num_cores=NC, num_subcores=NS)
    params = pltpu.CompilerParams(needs_layout_passes=False,
                                  use_tc_tiling_on_sc=True)

    detile = pl.kernel(
        _detile_body,
        out_type=jax.ShapeDtypeStruct((NE * D,), jnp.float32),
        mesh=mesh,
        compiler_params=params,
        scratch_types=[
            pltpu.VMEM((8, CW * 128), jnp.float32),
            pltpu.VMEM((CW * 128,), jnp.float32),
            pltpu.VMEM((TAIL_COLS, D), jnp.float32),
        ],
    )
    flat_ent = detile(rv_ent_embeds.T, rv_ent_embeds[FULL_TILES * 128:])

    out = jax.ShapeDtypeStruct((D, B), jnp.float32)
    run = pl.kernel(
        _gather_body,
        out_type=(out,) * 6,
        mesh=mesh,
        compiler_params=params,
        scratch_types=[
            pltpu.VMEM((BPW,), jnp.int32),
            pltpu.VMEM((D * BPW,), jnp.int32),
            pltpu.VMEM((D * BPW,), jnp.float32),
            pltpu.VMEM((BPW, D), jnp.float32),
            pltpu.VMEM((D, BPW), jnp.float32),
            pltpu.SemaphoreType.DMA,
        ],
    )
    outs = run(flat_ent, rel_embeds, rel_pos_hs, rel_pos_rs,
               rel_pos_ts, rel_neg_hs, rel_neg_rs, rel_neg_ts)
    return tuple(o.T for o in outs)


# speedup vs baseline: 1.2453x; 1.2453x over previous
import jax
import jax.numpy as jnp
from jax import lax
from jax.experimental import pallas as pl
from jax.experimental.pallas import tpu as pltpu
from jax.experimental.pallas import tpu_sc as plsc

D = 32
B = 16384
NC, NS, L = 2, 16, 16
NW = NC * NS
BPW = B // NW
CHUNKS = BPW // L


def _rsqrt_newton(s):
    i = plsc.bitcast(s, jnp.int32)
    i = jnp.int32(0x5F3759DF) - lax.shift_right_logical(i, 1)
    y = plsc.bitcast(i, jnp.float32)
    half_s = 0.5 * s
    for _ in range(3):
        y = y * (1.5 - half_s * y * y)
    return y


def _jobs_body(jobs, idx_v, land_v, out_v, sem):
    wid = lax.axis_index("s") * NC + lax.axis_index("c")
    base = wid * BPW

    lanes = lax.iota(jnp.int32, L)
    diag = [lax.bitwise_and(lanes + d, jnp.int32(D - 1)) for d in range(D)]

    for table, idx_hbm, out_hbm in jobs:
        pltpu.sync_copy(idx_hbm.at[pl.ds(base, BPW)], idx_v)

        def row_body(c, _):
            chunk = idx_v[pl.ds(c * L, L)]
            for jj in range(L):
                pltpu.async_copy(table.at[pl.ds(chunk[jj], 1)],
                                 land_v.at[pl.ds(c * L + jj, 1)], sem)
            return _

        lax.fori_loop(0, CHUNKS, row_body, None)
        pltpu.make_async_copy(table.at[pl.ds(0, BPW)], land_v, sem).wait()

        def norm_body(c, _):
            row_ids = c * L + lanes
            vals = [plsc.load_gather(land_v, [row_ids, diag[d]])
                    for d in range(D)]
            s = vals[0] * vals[0]
            for d in range(1, D):
                s = s + vals[d] * vals[d]
            y = _rsqrt_newton(jnp.maximum(s, 1e-24))
            for d in range(D):
                plsc.store_scatter(out_v, [diag[d], row_ids], vals[d] * y)
            return _

        lax.fori_loop(0, CHUNKS, norm_body, None)
        pltpu.sync_copy(out_v, out_hbm.at[:, pl.ds(base, BPW)])


def _ent_body(ent_hbm, ph, pt, nh, nt, o0, o2, o3, o5,
              idx_v, land_v, out_v, sem):
    _jobs_body(((ent_hbm, ph, o0), (ent_hbm, pt, o2),
                (ent_hbm, nh, o3), (ent_hbm, nt, o5)),
               idx_v, land_v, out_v, sem)


def _rel_body(rel_hbm, pr, nr, o1, o4, idx_v, land_v, out_v, sem):
    _jobs_body(((rel_hbm, pr, o1), (rel_hbm, nr, o4)),
               idx_v, land_v, out_v, sem)


_SCRATCH = [
    pltpu.VMEM((BPW,), jnp.int32),
    pltpu.VMEM((BPW, D), jnp.float32),
    pltpu.VMEM((D, BPW), jnp.float32),
    pltpu.SemaphoreType.DMA,
]


@jax.jit
def kernel(rv_ent_embeds, rel_embeds, rel_pos_hs, rel_pos_rs, rel_pos_ts,
           rel_neg_hs, rel_neg_rs, rel_neg_ts):
    out = jax.ShapeDtypeStruct((D, B), jnp.float32)
    mesh = plsc.VectorSubcoreMesh(core_axis_name="c", subcore_axis_name="s",
                                  num_cores=NC, num_subcores=NS)
    params = pltpu.CompilerParams(needs_layout_passes=False,
                                  use_tc_tiling_on_sc=True)
    run_rel = pl.kernel(_rel_body, out_type=(out,) * 2, mesh=mesh,
                        compiler_params=params, scratch_types=_SCRATCH)
    run_ent = pl.kernel(_ent_body, out_type=(out,) * 4, mesh=mesh,
                        compiler_params=params, scratch_types=_SCRATCH)
    o1, o4 = run_rel(rel_embeds, rel_pos_rs, rel_neg_rs)
    o0, o2, o3, o5 = run_ent(rv_ent_embeds, rel_pos_hs, rel_pos_ts,
                             rel_neg_hs, rel_neg_ts)
    return (o0.T, o1.T, o2.T, o3.T, o4.T, o5.T)


# speedup vs baseline: 1.2643x; 1.0153x over previous
import jax
import jax.numpy as jnp
from jax import lax
from jax.experimental import pallas as pl
from jax.experimental.pallas import tpu as pltpu
from jax.experimental.pallas import tpu_sc as plsc

D = 32
B = 16384
NC, NS, L = 2, 16, 16
NW = NC * NS
BPW = B // NW
CHUNKS = BPW // L


def _rsqrt_newton(s):
    i = plsc.bitcast(s, jnp.int32)
    i = jnp.int32(0x5F3759DF) - lax.shift_right_logical(i, 1)
    y = plsc.bitcast(i, jnp.float32)
    half_s = 0.5 * s
    for _ in range(3):
        y = y * (1.5 - half_s * y * y)
    return y


def _sc_body(ent_hbm, rel_hbm, ph, pr, pt, nh, nr, nt,
             o0, o1, o2, o3, o4, o5, idx_v, land_v, out_v, sem):
    wid = lax.axis_index("s") * NC + lax.axis_index("c")
    base = wid * BPW
    jobs = ((ent_hbm, ph, o0), (rel_hbm, pr, o1), (ent_hbm, pt, o2),
            (ent_hbm, nh, o3), (rel_hbm, nr, o4), (ent_hbm, nt, o5))

    lanes = lax.iota(jnp.int32, L)
    diag = [lax.bitwise_and(lanes + d, jnp.int32(D - 1)) for d in range(D)]

    for table, idx_hbm, out_hbm in jobs:
        pltpu.sync_copy(idx_hbm.at[pl.ds(base, BPW)], idx_v)

        def row_body(c, _):
            chunk = idx_v[pl.ds(c * L, L)]
            for jj in range(L):
                pltpu.async_copy(table.at[pl.ds(chunk[jj], 1)],
                                 land_v.at[pl.ds(c * L + jj, 1)], sem)
            return _

        lax.fori_loop(0, CHUNKS, row_body, None)
        pltpu.make_async_copy(table.at[pl.ds(0, BPW)], land_v, sem).wait()

        def norm_body(c, _):
            row_ids = c * L + lanes
            vals = [plsc.load_gather(land_v, [row_ids, diag[d]])
                    for d in range(D)]
            s = vals[0] * vals[0]
            for d in range(1, D):
                s = s + vals[d] * vals[d]
            y = _rsqrt_newton(jnp.maximum(s, 1e-24))
            for d in range(D):
                plsc.store_scatter(out_v, [diag[d], row_ids], vals[d] * y)
            return _

        lax.fori_loop(0, CHUNKS, norm_body, None)
        pltpu.sync_copy(out_v, out_hbm.at[:, pl.ds(base, BPW)])


@jax.jit
def kernel(rv_ent_embeds, rel_embeds, rel_pos_hs, rel_pos_rs, rel_pos_ts,
           rel_neg_hs, rel_neg_rs, rel_neg_ts):
    out = jax.ShapeDtypeStruct((D, B), jnp.float32)
    mesh = plsc.VectorSubcoreMesh(core_axis_name="c", subcore_axis_name="s",
                                  num_cores=NC, num_subcores=NS)
    run = pl.kernel(
        _sc_body,
        out_type=(out,) * 6,
        mesh=mesh,
        compiler_params=pltpu.CompilerParams(needs_layout_passes=False,
                                             use_tc_tiling_on_sc=True),
        scratch_types=[
            pltpu.VMEM((BPW,), jnp.int32),
            pltpu.VMEM((BPW, D), jnp.float32),
            pltpu.VMEM((D, BPW), jnp.float32),
            pltpu.SemaphoreType.DMA,
        ],
    )
    outs = run(rv_ent_embeds, rel_embeds, rel_pos_hs, rel_pos_rs,
               rel_pos_ts, rel_neg_hs, rel_neg_rs, rel_neg_ts)
    return tuple(o.T for o in outs)
